# baseline (speedup 1.0000x reference)
import functools
import math

import jax
import jax.numpy as jnp
from jax import lax
from jax.experimental import pallas as pl
from jax.experimental.pallas import tpu as pltpu
from jax.experimental.pallas import tpu_sc as plsc

VOCAB = 1000000
EMBED_DIM = 64
HARMONIC_BASES = 16
_HH = HARMONIC_BASES // 2

_B, _S = 1024, 50
_T = _B * _S
_NC, _NS = 2, 16
_NW = _NC * _NS
_TPW = _T // _NW
_EPW = _TPW * _HH

_W = 76928
_NB = 13
_NRH = _HH * _W // 128
_FLATH = _NB * _NRH * 128


def _detile_body(a_ref, p_ref, out_ref):
    a = a_ref[...].reshape(_NRH, 128).astype(jnp.bfloat16)
    p = p_ref[...].reshape(_NRH, 128).astype(jnp.bfloat16)
    a32 = lax.bitcast_convert_type(a, jnp.uint16).astype(jnp.int32)
    p32 = lax.bitcast_convert_type(p, jnp.uint16).astype(jnp.int32)
    out_ref[...] = a32 | (p32 << 16)


def _flatten_pair_half(tab_a, tab_p, half):
    q2 = pl.pallas_call(
        _detile_body,
        grid=(_NB,),
        in_specs=[
            pl.BlockSpec((_HH, _W), lambda j: (half, j)),
            pl.BlockSpec((_HH, _W), lambda j: (half, j)),
        ],
        out_specs=pl.BlockSpec((_NRH, 128), lambda j: (j, 0)),
        out_shape=jax.ShapeDtypeStruct((_NB * _NRH, 128), jnp.int32),
    )(tab_a.T, tab_p.T)
    return q2.reshape(_FLATH)


def _sc_gather_half(base, flat_ap, name):
    mesh = plsc.VectorSubcoreMesh(core_axis_name="c", subcore_axis_name="s")

    @functools.partial(
        pl.kernel,
        out_type=jax.ShapeDtypeStruct((_T * _HH,), jnp.int32),
        name=name,
        mesh=mesh,
        scratch_types=[
            pltpu.VMEM((_TPW,), jnp.int32),
            pltpu.VMEM((_EPW,), jnp.int32),
            pltpu.VMEM((_EPW,), jnp.int32),
            pltpu.SemaphoreType.DMA,
        ],
        compiler_params=pltpu.CompilerParams(use_tc_tiling_on_sc=False),
    )
    def gather_kernel(base_hbm, tab_hbm, out_hbm, base_v, ilist_v, vals, sem):
        wid = lax.axis_index("s") * _NC + lax.axis_index("c")
        tok0 = wid * _TPW
        pltpu.sync_copy(base_hbm.at[pl.ds(tok0, _TPW)], base_v)
        lanes = lax.iota(jnp.int32, 16)
        harm = (lanes & 7) * _W
        pair = lanes >> 3

        def build(k, carry):
            base16 = base_v[pl.ds(k * 16, 16)]
            for j in range(8):
                bj = base16[pair + 2 * j]
                ilist_v[pl.ds(k * 128 + j * 16, 16)] = bj + harm
            return carry

        lax.fori_loop(0, _TPW // 16, build, 0)
        pltpu.async_copy(tab_hbm.at[ilist_v], vals, sem).wait()
        pltpu.sync_copy(vals, out_hbm.at[pl.ds(wid * _EPW, _EPW)])

    return gather_kernel(base, flat_ap)


_BRH = 160


def _unpack(u):
    a = lax.bitcast_convert_type(
        (u & 0xFFFF).astype(jnp.uint16), jnp.bfloat16).astype(jnp.float32)
    p = lax.bitcast_convert_type(
        lax.shift_right_logical(u, 16).astype(jnp.uint16),
        jnp.bfloat16).astype(jnp.float32)
    return a, p


def _tc_body(ap0_ref, ap1_ref, sb0_ref, cb0_ref, sb1_ref, cb1_ref, out_ref):
    a0, p0 = _unpack(ap0_ref[...])
    a1, p1 = _unpack(ap1_ref[...])
    acc = jnp.dot(a0 * jnp.cos(p0), sb0_ref[...],
                  preferred_element_type=jnp.float32)
    acc += jnp.dot(a0 * jnp.sin(p0), cb0_ref[...],
                   preferred_element_type=jnp.float32)
    acc += jnp.dot(a1 * jnp.cos(p1), sb1_ref[...],
                   preferred_element_type=jnp.float32)
    acc += jnp.dot(a1 * jnp.sin(p1), cb1_ref[...],
                   preferred_element_type=jnp.float32)
    out_ref[...] = acc


def _tc_synth(ap0, ap1, sb0, cb0, sb1, cb1):
    nrow = _T // 16
    grid = (nrow // _BRH,)
    basis_spec = pl.BlockSpec((128, 16 * EMBED_DIM), lambda i: (0, 0))
    return pl.pallas_call(
        _tc_body,
        grid=grid,
        in_specs=[
            pl.BlockSpec((_BRH, 128), lambda i: (i, 0)),
            pl.BlockSpec((_BRH, 128), lambda i: (i, 0)),
            basis_spec, basis_spec, basis_spec, basis_spec,
        ],
        out_specs=pl.BlockSpec((_BRH, 16 * EMBED_DIM), lambda i: (i, 0)),
        out_shape=jax.ShapeDtypeStruct((nrow, 16 * EMBED_DIM), jnp.float32),
    )(ap0, ap1, sb0, cb0, sb1, cb1)


def kernel(x, frequency_amplitudes, frequency_phases, frequencies):
    idx = x.reshape(_T).astype(jnp.int32)
    base = idx + (idx // _W) * ((_HH - 1) * _W)

    flat0 = _flatten_pair_half(frequency_amplitudes, frequency_phases, 0)
    ap0_flat = _sc_gather_half(base, flat0, "sc_gather_h0")
    flat1 = _flatten_pair_half(frequency_amplitudes, frequency_phases, 1)
    ap1_flat = _sc_gather_half(base, flat1, "sc_gather_h1")
    ap0 = ap0_flat.reshape(_T // 16, 128)
    ap1 = ap1_flat.reshape(_T // 16, 128)

    t = jnp.linspace(0.0, 1.0, EMBED_DIM, dtype=jnp.float32)
    theta = (2.0 * math.pi) * frequencies[:, None] * t[None, :]
    eye16 = jnp.eye(16, dtype=jnp.float32)
    sb0 = jnp.kron(eye16, jnp.sin(theta[:_HH]))
    cb0 = jnp.kron(eye16, jnp.cos(theta[:_HH]))
    sb1 = jnp.kron(eye16, jnp.sin(theta[_HH:]))
    cb1 = jnp.kron(eye16, jnp.cos(theta[_HH:]))

    out = _tc_synth(ap0, ap1, sb0, cb0, sb1, cb1)
    return out.reshape(_B, _S, EMBED_DIM)


# speedup vs baseline: 1.0056x; 1.0056x over previous
import functools
import math

import jax
import jax.numpy as jnp
from jax import lax
from jax.experimental import pallas as pl
from jax.experimental.pallas import tpu as pltpu
from jax.experimental.pallas import tpu_sc as plsc

VOCAB = 1000000
EMBED_DIM = 64
HARMONIC_BASES = 16

_B, _S = 1024, 50
_T = _B * _S
_NC, _NS = 2, 16
_NW = _NC * _NS
_TPW = _T // _NW
_EPW = _TPW * HARMONIC_BASES
_PR = _T // 8

_W = 76928
_NB = 13
_NR = HARMONIC_BASES * _W // 128
_FLAT = _NB * _NR * 128


def _detile_body(a_ref, p_ref, out_ref):
    a = a_ref[...].reshape(_NR, 128).astype(jnp.bfloat16)
    p = p_ref[...].reshape(_NR, 128).astype(jnp.bfloat16)
    a32 = lax.bitcast_convert_type(a, jnp.uint16).astype(jnp.int32)
    p32 = lax.bitcast_convert_type(p, jnp.uint16).astype(jnp.int32)
    out_ref[...] = a32 | (p32 << 16)


def _flatten_pair(tab_a, tab_p):
    q2 = pl.pallas_call(
        _detile_body,
        grid=(_NB,),
        in_specs=[
            pl.BlockSpec((HARMONIC_BASES, _W), lambda j: (0, j)),
            pl.BlockSpec((HARMONIC_BASES, _W), lambda j: (0, j)),
        ],
        out_specs=pl.BlockSpec((_NR, 128), lambda j: (j, 0)),
        out_shape=jax.ShapeDtypeStruct((_NB * _NR, 128), jnp.int32),
    )(tab_a.T, tab_p.T)
    return q2.reshape(_FLAT)


def _sc_gather(base, flat_ap):
    mesh = plsc.VectorSubcoreMesh(core_axis_name="c", subcore_axis_name="s")

    @functools.partial(
        pl.kernel,
        out_type=jax.ShapeDtypeStruct((_T * HARMONIC_BASES,), jnp.int32),
        name="sc_spectral_gather",
        mesh=mesh,
        scratch_types=[
            pltpu.VMEM((_TPW,), jnp.int32),
            pltpu.VMEM((_EPW,), jnp.int32),
            pltpu.VMEM((_EPW,), jnp.int32),
            pltpu.SemaphoreType.DMA,
        ],
        compiler_params=pltpu.CompilerParams(use_tc_tiling_on_sc=False),
    )
    def gather_kernel(base_hbm, tab_hbm, out_hbm, base_v, ilist_v, vals, sem):
        wid = lax.axis_index("s") * _NC + lax.axis_index("c")
        tok0 = wid * _TPW
        pltpu.sync_copy(base_hbm.at[pl.ds(tok0, _TPW)], base_v)
        harm = lax.iota(jnp.int32, 16) * _W

        def build(k, carry):
            base16 = base_v[pl.ds(k * 16, 16)]
            for j in range(16):
                bj = base16[jnp.full((16,), j, jnp.int32)]
                ilist_v[pl.ds((k * 16 + j) * 16, 16)] = bj + harm
            return carry

        lax.fori_loop(0, _TPW // 16, build, 0)
        pltpu.async_copy(tab_hbm.at[ilist_v], vals, sem).wait()
        pltpu.sync_copy(vals, out_hbm.at[pl.ds(wid * _EPW, _EPW)])

    return gather_kernel(base, flat_ap)


_BR = 320


def _tc_body(ap_ref, sb_ref, cb_ref, out_ref):
    u = ap_ref[...]
    a = lax.bitcast_convert_type(
        (u & 0xFFFF).astype(jnp.uint16), jnp.bfloat16).astype(jnp.float32)
    p = lax.bitcast_convert_type(
        lax.shift_right_logical(u, 16).astype(jnp.uint16),
        jnp.bfloat16).astype(jnp.float32)
    w = a * jnp.cos(p)
    z = a * jnp.sin(p)
    out_ref[...] = (
        jnp.dot(w, sb_ref[...], preferred_element_type=jnp.float32)
        + jnp.dot(z, cb_ref[...], preferred_element_type=jnp.float32)
    )


def _tc_synth(ap_packed, sb, cb):
    grid = (_PR // _BR,)
    return pl.pallas_call(
        _tc_body,
        grid=grid,
        in_specs=[
            pl.BlockSpec((_BR, 128), lambda i: (i, 0)),
            pl.BlockSpec((128, 8 * EMBED_DIM), lambda i: (0, 0)),
            pl.BlockSpec((128, 8 * EMBED_DIM), lambda i: (0, 0)),
        ],
        out_specs=pl.BlockSpec((_BR, 8 * EMBED_DIM), lambda i: (i, 0)),
        out_shape=jax.ShapeDtypeStruct((_PR, 8 * EMBED_DIM), jnp.float32),
    )(ap_packed, sb, cb)


def kernel(x, frequency_amplitudes, frequency_phases, frequencies):
    idx = x.reshape(_T).astype(jnp.int32)
    base = idx + (idx // _W) * ((HARMONIC_BASES - 1) * _W)
    flat_ap = _flatten_pair(frequency_amplitudes, frequency_phases)
    ap_flat = _sc_gather(base, flat_ap)
    ap_packed = ap_flat.reshape(_PR, 128)

    t = jnp.linspace(0.0, 1.0, EMBED_DIM, dtype=jnp.float32)
    theta = (2.0 * math.pi) * frequencies[:, None] * t[None, :]
    eye8 = jnp.eye(8, dtype=jnp.float32)
    sb = jnp.kron(eye8, jnp.sin(theta))
    cb = jnp.kron(eye8, jnp.cos(theta))

    out = _tc_synth(ap_packed, sb, cb)
    return out.reshape(_B, _S, EMBED_DIM)
